# Initial kernel scaffold; baseline (speedup 1.0000x reference)
import jax
import jax.numpy as jnp
from jax import lax
from jax.experimental import pallas as pl

_B, _A, _NBH = 10, 1000, 32
_D = 128
_NG = 25
_NI = 3
_NFB = 3
_CUTOFF = 5.0
_MAXZ = 100

_AP = 1024
_CA = 64
_NC = _AP // _CA
_E = _CA * _NBH
_NGP = 32


def _emb_kernel(az_ref, ehi_ref, elo_ref, x_ref):
    az = az_ref[0]
    iot = lax.broadcasted_iota(jnp.float32, (_AP, 128), 1)
    ohz = (iot == az).astype(jnp.bfloat16)
    x = jnp.dot(ohz, ehi_ref[...], preferred_element_type=jnp.float32)
    x = x + jnp.dot(ohz, elo_ref[...], preferred_element_type=jnp.float32)
    x_ref[0] = x


def _fij_kernel(nbr_ref, slf_ref, p_ref, fij_ref, c_ref):
    nbr = nbr_ref[0, 0]
    slf = slf_ref[0]
    iot = lax.broadcasted_iota(jnp.float32, (_E, _AP), 1)
    oh = (iot == nbr).astype(jnp.bfloat16) - (iot == slf).astype(jnp.bfloat16)
    dall = jnp.dot(oh, p_ref[0], preferred_element_type=jnp.float32)
    r2 = jnp.zeros((_E, 1), jnp.float32)
    for c in range(3):
        dv = dall[:, c:c + 1] + dall[:, c + 4:c + 5]
        r2 = r2 + dv * dv
    r = jnp.sqrt(r2)
    width = _CUTOFF / (_NG - 1)
    coeff = -0.5 / (width * width)
    offs = lax.broadcasted_iota(jnp.float32, (_E, _NGP), 1) * width
    fij_ref[0, 0] = jnp.exp(coeff * (r - offs) ** 2)
    c_ref[0, 0] = (r <= _CUTOFF).astype(jnp.float32)


def _y_kernel(x_ref, w_ref, b_ref, y_ref):
    y = jnp.dot(x_ref[0], w_ref[...], preferred_element_type=jnp.float32)
    y_ref[0] = (y + b_ref[...]).astype(jnp.bfloat16)


def _main_kernel(fij_ref, c_ref, nbr_ref, y_ref, x_ref,
                 win_ref, bin_ref, wh0_ref, bh0_ref, wh1_ref, bh1_ref,
                 wh2_ref, bh2_ref, f2w_ref, f2b_ref, dw_ref, db_ref, xo_ref):
    gelu = jax.nn.gelu
    fij = fij_ref[0, 0]
    w = gelu(jnp.dot(fij, win_ref[...], preferred_element_type=jnp.float32)
             + bin_ref[...])
    for wh_ref, bh_ref in ((wh0_ref, bh0_ref), (wh1_ref, bh1_ref),
                           (wh2_ref, bh2_ref)):
        w = gelu(jnp.dot(w, wh_ref[...], preferred_element_type=jnp.float32)
                 + bh_ref[...])
    w = w * c_ref[0, 0]
    nbr = nbr_ref[0, 0]
    iot = lax.broadcasted_iota(jnp.float32, (_E, _AP), 1)
    oh = (iot == nbr).astype(jnp.bfloat16)
    yj = jnp.dot(oh, y_ref[0], preferred_element_type=jnp.float32)
    prod = yj * w
    yagg = prod[0:_CA, :]
    for k in range(1, _NBH):
        yagg = yagg + prod[k * _CA:(k + 1) * _CA, :]
    t = gelu(jnp.dot(yagg, f2w_ref[...], preferred_element_type=jnp.float32)
             + f2b_ref[...])
    v = jnp.dot(t, dw_ref[...], preferred_element_type=jnp.float32) + db_ref[...]
    xo_ref[0] = x_ref[0] + v


def _full(shape):
    return pl.BlockSpec(shape, lambda *_: tuple(0 for _ in shape))


def kernel(atomic_numbers, positions, cell, cell_offset, neighbors,
           neighbor_mask, atom_mask, emb, filt_Win, filt_bin, filt_Wh,
           filt_bh, in2f_W, in2f_b, f2out_W, f2out_b, dense_W, dense_b):
    f32, bf16 = jnp.float32, jnp.bfloat16
    pada = _AP - _A
    az = jnp.pad(atomic_numbers, ((0, 0), (0, pada))).astype(f32)[..., None]
    pos = jnp.pad(positions, ((0, 0), (0, pada), (0, 0)))
    phi = pos.astype(bf16)
    plo = (pos - phi.astype(f32)).astype(bf16)
    ptab = jnp.concatenate(
        [phi, jnp.zeros((_B, _AP, 1), bf16), plo,
         jnp.zeros((_B, _AP, _D - 7), bf16)], axis=-1)
    nbr = jnp.pad(neighbors, ((0, 0), (0, pada), (0, 0)))
    nbr_k = (nbr.reshape(_B, _NC, _CA, _NBH).transpose(0, 1, 3, 2)
             .reshape(_B, _NC, _E, 1).astype(f32))
    slf = (jnp.arange(_NC, dtype=f32)[:, None] * _CA
           + jnp.tile(jnp.arange(_CA, dtype=f32), _NBH)[None, :]
           ).reshape(_NC, _E, 1)
    ehi16 = emb.astype(bf16)
    ehi = jnp.zeros((128, _D), bf16).at[:_MAXZ].set(ehi16)
    elo = jnp.zeros((128, _D), bf16).at[:_MAXZ].set(
        (emb - ehi16.astype(f32)).astype(bf16))
    winp = jnp.pad(filt_Win, ((0, 0), (0, _NGP - _NG), (0, 0)))

    x = pl.pallas_call(
        _emb_kernel, grid=(_B,),
        in_specs=[pl.BlockSpec((1, _AP, 1), lambda b: (b, 0, 0)),
                  _full((128, _D)), _full((128, _D))],
        out_specs=pl.BlockSpec((1, _AP, _D), lambda b: (b, 0, 0)),
        out_shape=jax.ShapeDtypeStruct((_B, _AP, _D), f32),
    )(az, ehi, elo)

    fij, cmask = pl.pallas_call(
        _fij_kernel, grid=(_B, _NC),
        in_specs=[pl.BlockSpec((1, 1, _E, 1), lambda b, c: (b, c, 0, 0)),
                  pl.BlockSpec((1, _E, 1), lambda b, c: (c, 0, 0)),
                  pl.BlockSpec((1, _AP, _D), lambda b, c: (b, 0, 0))],
        out_specs=[pl.BlockSpec((1, 1, _E, _NGP), lambda b, c: (b, c, 0, 0)),
                   pl.BlockSpec((1, 1, _E, 1), lambda b, c: (b, c, 0, 0))],
        out_shape=[jax.ShapeDtypeStruct((_B, _NC, _E, _NGP), f32),
                   jax.ShapeDtypeStruct((_B, _NC, _E, 1), f32)],
    )(nbr_k, slf, ptab)

    for i in range(_NI):
        yb = pl.pallas_call(
            _y_kernel, grid=(_B,),
            in_specs=[pl.BlockSpec((1, _AP, _D), lambda b: (b, 0, 0)),
                      _full((_D, _D)), _full((1, _D))],
            out_specs=pl.BlockSpec((1, _AP, _D), lambda b: (b, 0, 0)),
            out_shape=jax.ShapeDtypeStruct((_B, _AP, _D), bf16),
        )(x, in2f_W[i], in2f_b[i].reshape(1, _D))

        x = pl.pallas_call(
            _main_kernel, grid=(_B, _NC),
            in_specs=[
                pl.BlockSpec((1, 1, _E, _NGP), lambda b, c: (b, c, 0, 0)),
                pl.BlockSpec((1, 1, _E, 1), lambda b, c: (b, c, 0, 0)),
                pl.BlockSpec((1, 1, _E, 1), lambda b, c: (b, c, 0, 0)),
                pl.BlockSpec((1, _AP, _D), lambda b, c: (b, 0, 0)),
                pl.BlockSpec((1, _CA, _D), lambda b, c: (b, c, 0)),
                _full((_NGP, _D)), _full((1, _D)),
                _full((_D, _D)), _full((1, _D)),
                _full((_D, _D)), _full((1, _D)),
                _full((_D, _D)), _full((1, _D)),
                _full((_D, _D)), _full((1, _D)),
                _full((_D, _D)), _full((1, _D)),
            ],
            out_specs=pl.BlockSpec((1, _CA, _D), lambda b, c: (b, c, 0)),
            out_shape=jax.ShapeDtypeStruct((_B, _AP, _D), f32),
        )(fij, cmask, nbr_k, yb, x,
          winp[i], filt_bin[i].reshape(1, _D),
          filt_Wh[i, 0], filt_bh[i, 0].reshape(1, _D),
          filt_Wh[i, 1], filt_bh[i, 1].reshape(1, _D),
          filt_Wh[i, 2], filt_bh[i, 2].reshape(1, _D),
          f2out_W[i], f2out_b[i].reshape(1, _D),
          dense_W[i], dense_b[i].reshape(1, _D))

    return x[:, :_A, :]


# speedup vs baseline: 7.4599x; 7.4599x over previous
import jax
import jax.numpy as jnp
from jax import lax
from jax.experimental import pallas as pl

_B, _A, _NBH = 10, 1000, 32
_D = 128
_NG = 25
_NI = 3
_NFB = 3
_CUTOFF = 5.0
_MAXZ = 100

_AP = 1024
_CA = 64
_NC = _AP // _CA
_E = _CA * _NBH
_NGP = 32


def _emb_kernel(az_ref, ehi_ref, elo_ref, x_ref):
    az = az_ref[0]
    iot = lax.broadcasted_iota(jnp.int32, (_AP, 128), 1)
    ohz = (iot == az).astype(jnp.bfloat16)
    x = jnp.dot(ohz, ehi_ref[...], preferred_element_type=jnp.float32)
    x = x + jnp.dot(ohz, elo_ref[...], preferred_element_type=jnp.float32)
    x_ref[0] = x


def _fij_kernel(nbr_ref, slf_ref, p_ref, fij_ref, c_ref):
    nbr = nbr_ref[0, 0]
    slf = slf_ref[0]
    iot = lax.broadcasted_iota(jnp.int32, (_E, _AP), 1)
    oh = (iot == nbr).astype(jnp.bfloat16) - (iot == slf).astype(jnp.bfloat16)
    dall = jnp.dot(oh, p_ref[0], preferred_element_type=jnp.float32)
    r2 = jnp.zeros((_E, 1), jnp.float32)
    for c in range(3):
        dv = dall[:, c:c + 1] + dall[:, c + 4:c + 5]
        r2 = r2 + dv * dv
    r = jnp.sqrt(r2)
    width = _CUTOFF / (_NG - 1)
    coeff = -0.5 / (width * width)
    offs = lax.broadcasted_iota(jnp.int32, (_E, _NGP), 1).astype(jnp.float32) * width
    fij_ref[0, 0] = jnp.exp(coeff * (r - offs) ** 2)
    c_ref[0, 0] = (r <= _CUTOFF).astype(jnp.float32)


def _y_kernel(x_ref, w_ref, b_ref, y_ref):
    y = jnp.dot(x_ref[0], w_ref[...], preferred_element_type=jnp.float32)
    y_ref[0] = (y + b_ref[...]).astype(jnp.bfloat16)


def _main_kernel(fij_ref, c_ref, nbr_ref, y_ref, x_ref,
                 win_ref, bin_ref, wh0_ref, bh0_ref, wh1_ref, bh1_ref,
                 wh2_ref, bh2_ref, f2w_ref, f2b_ref, dw_ref, db_ref, xo_ref):
    gelu = jax.nn.gelu
    fij = fij_ref[0, 0]
    w = gelu(jnp.dot(fij, win_ref[...], preferred_element_type=jnp.float32)
             + bin_ref[...])
    for wh_ref, bh_ref in ((wh0_ref, bh0_ref), (wh1_ref, bh1_ref),
                           (wh2_ref, bh2_ref)):
        w = gelu(jnp.dot(w, wh_ref[...], preferred_element_type=jnp.float32)
                 + bh_ref[...])
    w = w * c_ref[0, 0]
    nbr = nbr_ref[0, 0]
    iot = lax.broadcasted_iota(jnp.int32, (_E, _AP), 1)
    oh = (iot == nbr).astype(jnp.bfloat16)
    yj = jnp.dot(oh, y_ref[0], preferred_element_type=jnp.float32)
    prod = yj * w
    yagg = prod[0:_CA, :]
    for k in range(1, _NBH):
        yagg = yagg + prod[k * _CA:(k + 1) * _CA, :]
    t = gelu(jnp.dot(yagg, f2w_ref[...], preferred_element_type=jnp.float32)
             + f2b_ref[...])
    v = jnp.dot(t, dw_ref[...], preferred_element_type=jnp.float32) + db_ref[...]
    xo_ref[0] = x_ref[0] + v


def _full(shape):
    return pl.BlockSpec(shape, lambda *_: tuple(0 for _ in shape))


def kernel(atomic_numbers, positions, cell, cell_offset, neighbors,
           neighbor_mask, atom_mask, emb, filt_Win, filt_bin, filt_Wh,
           filt_bh, in2f_W, in2f_b, f2out_W, f2out_b, dense_W, dense_b):
    f32, bf16 = jnp.float32, jnp.bfloat16
    pada = _AP - _A
    az = jnp.pad(atomic_numbers, ((0, 0), (0, pada))).astype(jnp.int32)[..., None]
    pos = jnp.pad(positions, ((0, 0), (0, pada), (0, 0)))
    phi = pos.astype(bf16)
    plo = (pos - phi.astype(f32)).astype(bf16)
    ptab = jnp.concatenate(
        [phi, jnp.zeros((_B, _AP, 1), bf16), plo,
         jnp.zeros((_B, _AP, _D - 7), bf16)], axis=-1)
    nbr = jnp.pad(neighbors, ((0, 0), (0, pada), (0, 0)))
    nbr_k = (nbr.reshape(_B, _NC, _CA, _NBH).transpose(0, 1, 3, 2)
             .reshape(_B, _NC, _E, 1).astype(jnp.int32))
    slf = (jnp.arange(_NC, dtype=jnp.int32)[:, None] * _CA
           + jnp.tile(jnp.arange(_CA, dtype=jnp.int32), _NBH)[None, :]
           ).reshape(_NC, _E, 1)
    ehi16 = emb.astype(bf16)
    ehi = jnp.zeros((128, _D), bf16).at[:_MAXZ].set(ehi16)
    elo = jnp.zeros((128, _D), bf16).at[:_MAXZ].set(
        (emb - ehi16.astype(f32)).astype(bf16))
    winp = jnp.pad(filt_Win, ((0, 0), (0, _NGP - _NG), (0, 0)))

    x = pl.pallas_call(
        _emb_kernel, grid=(_B,),
        in_specs=[pl.BlockSpec((1, _AP, 1), lambda b: (b, 0, 0)),
                  _full((128, _D)), _full((128, _D))],
        out_specs=pl.BlockSpec((1, _AP, _D), lambda b: (b, 0, 0)),
        out_shape=jax.ShapeDtypeStruct((_B, _AP, _D), f32),
    )(az, ehi, elo)

    fij, cmask = pl.pallas_call(
        _fij_kernel, grid=(_B, _NC),
        in_specs=[pl.BlockSpec((1, 1, _E, 1), lambda b, c: (b, c, 0, 0)),
                  pl.BlockSpec((1, _E, 1), lambda b, c: (c, 0, 0)),
                  pl.BlockSpec((1, _AP, _D), lambda b, c: (b, 0, 0))],
        out_specs=[pl.BlockSpec((1, 1, _E, _NGP), lambda b, c: (b, c, 0, 0)),
                   pl.BlockSpec((1, 1, _E, 1), lambda b, c: (b, c, 0, 0))],
        out_shape=[jax.ShapeDtypeStruct((_B, _NC, _E, _NGP), f32),
                   jax.ShapeDtypeStruct((_B, _NC, _E, 1), f32)],
    )(nbr_k, slf, ptab)

    for i in range(_NI):
        yb = pl.pallas_call(
            _y_kernel, grid=(_B,),
            in_specs=[pl.BlockSpec((1, _AP, _D), lambda b: (b, 0, 0)),
                      _full((_D, _D)), _full((1, _D))],
            out_specs=pl.BlockSpec((1, _AP, _D), lambda b: (b, 0, 0)),
            out_shape=jax.ShapeDtypeStruct((_B, _AP, _D), bf16),
        )(x, in2f_W[i], in2f_b[i].reshape(1, _D))

        x = pl.pallas_call(
            _main_kernel, grid=(_B, _NC),
            in_specs=[
                pl.BlockSpec((1, 1, _E, _NGP), lambda b, c: (b, c, 0, 0)),
                pl.BlockSpec((1, 1, _E, 1), lambda b, c: (b, c, 0, 0)),
                pl.BlockSpec((1, 1, _E, 1), lambda b, c: (b, c, 0, 0)),
                pl.BlockSpec((1, _AP, _D), lambda b, c: (b, 0, 0)),
                pl.BlockSpec((1, _CA, _D), lambda b, c: (b, c, 0)),
                _full((_NGP, _D)), _full((1, _D)),
                _full((_D, _D)), _full((1, _D)),
                _full((_D, _D)), _full((1, _D)),
                _full((_D, _D)), _full((1, _D)),
                _full((_D, _D)), _full((1, _D)),
                _full((_D, _D)), _full((1, _D)),
            ],
            out_specs=pl.BlockSpec((1, _CA, _D), lambda b, c: (b, c, 0)),
            out_shape=jax.ShapeDtypeStruct((_B, _AP, _D), f32),
        )(fij, cmask, nbr_k, yb, x,
          winp[i], filt_bin[i].reshape(1, _D),
          filt_Wh[i, 0], filt_bh[i, 0].reshape(1, _D),
          filt_Wh[i, 1], filt_bh[i, 1].reshape(1, _D),
          filt_Wh[i, 2], filt_bh[i, 2].reshape(1, _D),
          f2out_W[i], f2out_b[i].reshape(1, _D),
          dense_W[i], dense_b[i].reshape(1, _D))

    return x[:, :_A, :]


# speedup vs baseline: 8.4923x; 1.1384x over previous
import jax
import jax.numpy as jnp
from jax import lax
from jax.experimental import pallas as pl

_B, _A, _NBH = 10, 1000, 32
_D = 128
_NG = 25
_NI = 3
_NFB = 3
_CUTOFF = 5.0
_MAXZ = 100

_AP = 1024
_CA = 128
_NC = _AP // _CA
_E = _CA * _NBH
_NGP = 32


def _emb_kernel(az_ref, ehi_ref, elo_ref, x_ref):
    az = az_ref[0]
    iot = lax.broadcasted_iota(jnp.int32, (_AP, 128), 1)
    ohz = (iot == az).astype(jnp.bfloat16)
    x = jnp.dot(ohz, ehi_ref[...], preferred_element_type=jnp.float32)
    x = x + jnp.dot(ohz, elo_ref[...], preferred_element_type=jnp.float32)
    x_ref[0] = x


def _fij_kernel(nbr_ref, p_ref, pc_ref, fij_ref, c_ref):
    nbr = nbr_ref[0, 0]
    iot = lax.broadcasted_iota(jnp.int32, (_E, _AP), 1)
    oh = (iot == nbr).astype(jnp.bfloat16)
    dall = jnp.dot(oh, p_ref[0], preferred_element_type=jnp.float32)
    pc = pc_ref[0].astype(jnp.float32)
    dall = dall - jnp.concatenate([pc] * _NBH, axis=0)
    r2 = jnp.zeros((_E, 1), jnp.float32)
    for c in range(3):
        dv = dall[:, c:c + 1] + dall[:, c + 4:c + 5]
        r2 = r2 + dv * dv
    r = jnp.sqrt(r2)
    width = _CUTOFF / (_NG - 1)
    coeff = -0.5 / (width * width)
    offs = lax.broadcasted_iota(jnp.int32, (_E, _NGP), 1).astype(jnp.float32) * width
    fij_ref[0, 0] = jnp.exp(coeff * (r - offs) ** 2).astype(jnp.bfloat16)
    c_ref[0, 0] = (r <= _CUTOFF).astype(jnp.float32)


def _y_kernel(x_ref, w_ref, b_ref, y_ref):
    y = jnp.dot(x_ref[0], w_ref[...], preferred_element_type=jnp.float32)
    y_ref[0] = (y + b_ref[...]).astype(jnp.bfloat16)


def _main_kernel(fij_ref, c_ref, nbr_ref, y_ref, x_ref,
                 win_ref, bin_ref, wh0_ref, bh0_ref, wh1_ref, bh1_ref,
                 wh2_ref, bh2_ref, f2w_ref, f2b_ref, dw_ref, db_ref, xo_ref):
    gelu = jax.nn.gelu
    bf16 = jnp.bfloat16
    fij = fij_ref[0, 0]
    w = gelu(jnp.dot(fij, win_ref[...], preferred_element_type=jnp.float32)
             + bin_ref[...])
    for wh_ref, bh_ref in ((wh0_ref, bh0_ref), (wh1_ref, bh1_ref),
                           (wh2_ref, bh2_ref)):
        w = gelu(jnp.dot(w.astype(bf16), wh_ref[...],
                         preferred_element_type=jnp.float32) + bh_ref[...])
    w = w * c_ref[0, 0]
    nbr = nbr_ref[0, 0]
    iot = lax.broadcasted_iota(jnp.int32, (_E, _AP), 1)
    oh = (iot == nbr).astype(bf16)
    yj = jnp.dot(oh, y_ref[0], preferred_element_type=jnp.float32)
    prod = yj * w
    parts = [prod[k * _CA:(k + 1) * _CA, :] for k in range(_NBH)]
    while len(parts) > 1:
        parts = [parts[j] + parts[j + 1] for j in range(0, len(parts), 2)]
    yagg = parts[0]
    t = gelu(jnp.dot(yagg, f2w_ref[...], preferred_element_type=jnp.float32)
             + f2b_ref[...])
    v = jnp.dot(t, dw_ref[...], preferred_element_type=jnp.float32) + db_ref[...]
    xo_ref[0] = x_ref[0] + v


def _full(shape):
    return pl.BlockSpec(shape, lambda *_: tuple(0 for _ in shape))


def kernel(atomic_numbers, positions, cell, cell_offset, neighbors,
           neighbor_mask, atom_mask, emb, filt_Win, filt_bin, filt_Wh,
           filt_bh, in2f_W, in2f_b, f2out_W, f2out_b, dense_W, dense_b):
    f32, bf16 = jnp.float32, jnp.bfloat16
    pada = _AP - _A
    az = jnp.pad(atomic_numbers, ((0, 0), (0, pada))).astype(jnp.int32)[..., None]
    pos = jnp.pad(positions, ((0, 0), (0, pada), (0, 0)))
    phi = pos.astype(bf16)
    plo = (pos - phi.astype(f32)).astype(bf16)
    ptab = jnp.concatenate(
        [phi, jnp.zeros((_B, _AP, 1), bf16), plo,
         jnp.zeros((_B, _AP, _D - 7), bf16)], axis=-1)
    nbr = jnp.pad(neighbors, ((0, 0), (0, pada), (0, 0)))
    nbr_k = (nbr.reshape(_B, _NC, _CA, _NBH).transpose(0, 1, 3, 2)
             .reshape(_B, _NC, _E, 1).astype(jnp.int32))
    ehi16 = emb.astype(bf16)
    ehi = jnp.zeros((128, _D), bf16).at[:_MAXZ].set(ehi16)
    elo = jnp.zeros((128, _D), bf16).at[:_MAXZ].set(
        (emb - ehi16.astype(f32)).astype(bf16))
    winp = jnp.pad(filt_Win, ((0, 0), (0, _NGP - _NG), (0, 0))).astype(bf16)
    wh16 = filt_Wh.astype(bf16)

    x = pl.pallas_call(
        _emb_kernel, grid=(_B,),
        in_specs=[pl.BlockSpec((1, _AP, 1), lambda b: (b, 0, 0)),
                  _full((128, _D)), _full((128, _D))],
        out_specs=pl.BlockSpec((1, _AP, _D), lambda b: (b, 0, 0)),
        out_shape=jax.ShapeDtypeStruct((_B, _AP, _D), f32),
    )(az, ehi, elo)

    fij, cmask = pl.pallas_call(
        _fij_kernel, grid=(_B, _NC),
        in_specs=[pl.BlockSpec((1, 1, _E, 1), lambda b, c: (b, c, 0, 0)),
                  pl.BlockSpec((1, _AP, _D), lambda b, c: (b, 0, 0)),
                  pl.BlockSpec((1, _CA, _D), lambda b, c: (b, c, 0))],
        out_specs=[pl.BlockSpec((1, 1, _E, _NGP), lambda b, c: (b, c, 0, 0)),
                   pl.BlockSpec((1, 1, _E, 1), lambda b, c: (b, c, 0, 0))],
        out_shape=[jax.ShapeDtypeStruct((_B, _NC, _E, _NGP), bf16),
                   jax.ShapeDtypeStruct((_B, _NC, _E, 1), f32)],
    )(nbr_k, ptab, ptab)

    for i in range(_NI):
        yb = pl.pallas_call(
            _y_kernel, grid=(_B,),
            in_specs=[pl.BlockSpec((1, _AP, _D), lambda b: (b, 0, 0)),
                      _full((_D, _D)), _full((1, _D))],
            out_specs=pl.BlockSpec((1, _AP, _D), lambda b: (b, 0, 0)),
            out_shape=jax.ShapeDtypeStruct((_B, _AP, _D), bf16),
        )(x, in2f_W[i], in2f_b[i].reshape(1, _D))

        x = pl.pallas_call(
            _main_kernel, grid=(_B, _NC),
            in_specs=[
                pl.BlockSpec((1, 1, _E, _NGP), lambda b, c: (b, c, 0, 0)),
                pl.BlockSpec((1, 1, _E, 1), lambda b, c: (b, c, 0, 0)),
                pl.BlockSpec((1, 1, _E, 1), lambda b, c: (b, c, 0, 0)),
                pl.BlockSpec((1, _AP, _D), lambda b, c: (b, 0, 0)),
                pl.BlockSpec((1, _CA, _D), lambda b, c: (b, c, 0)),
                _full((_NGP, _D)), _full((1, _D)),
                _full((_D, _D)), _full((1, _D)),
                _full((_D, _D)), _full((1, _D)),
                _full((_D, _D)), _full((1, _D)),
                _full((_D, _D)), _full((1, _D)),
                _full((_D, _D)), _full((1, _D)),
            ],
            out_specs=pl.BlockSpec((1, _CA, _D), lambda b, c: (b, c, 0)),
            out_shape=jax.ShapeDtypeStruct((_B, _AP, _D), f32),
        )(fij, cmask, nbr_k, yb, x,
          winp[i], filt_bin[i].reshape(1, _D),
          wh16[i, 0], filt_bh[i, 0].reshape(1, _D),
          wh16[i, 1], filt_bh[i, 1].reshape(1, _D),
          wh16[i, 2], filt_bh[i, 2].reshape(1, _D),
          f2out_W[i], f2out_b[i].reshape(1, _D),
          dense_W[i], dense_b[i].reshape(1, _D))

    return x[:, :_A, :]


# speedup vs baseline: 10.1526x; 1.1955x over previous
import jax
import jax.numpy as jnp
from jax import lax
from jax.experimental import pallas as pl
from jax.experimental.pallas import tpu as pltpu

_B, _A, _NBH = 10, 1000, 32
_D = 128
_NG = 25
_NI = 3
_NFB = 3
_CUTOFF = 5.0
_MAXZ = 100

_AP = 1024
_CA = 256
_NC = _AP // _CA
_E = _CA * _NBH
_NGP = 32


def _emb_kernel(az_ref, ehi_ref, elo_ref, x_ref):
    az = az_ref[0]
    iot = lax.broadcasted_iota(jnp.int32, (_AP, 128), 1)
    ohz = (iot == az).astype(jnp.bfloat16)
    x = jnp.dot(ohz, ehi_ref[...], preferred_element_type=jnp.float32)
    x = x + jnp.dot(ohz, elo_ref[...], preferred_element_type=jnp.float32)
    x_ref[0] = x


def _fij_kernel(nbr_ref, p_ref, pc_ref, fij_ref, c_ref):
    nbr = nbr_ref[0, 0]
    iot = lax.broadcasted_iota(jnp.int32, (_E, _AP), 1)
    oh = (iot == nbr).astype(jnp.bfloat16)
    dall = jnp.dot(oh, p_ref[0], preferred_element_type=jnp.float32)
    pc = pc_ref[0].astype(jnp.float32)
    dall = dall - jnp.concatenate([pc] * _NBH, axis=0)
    r2 = jnp.zeros((_E, 1), jnp.float32)
    for c in range(3):
        dv = dall[:, c:c + 1] + dall[:, c + 4:c + 5]
        r2 = r2 + dv * dv
    r = jnp.sqrt(r2)
    width = _CUTOFF / (_NG - 1)
    coeff = -0.5 / (width * width)
    offs = lax.broadcasted_iota(jnp.int32, (_E, _NGP), 1).astype(jnp.float32) * width
    fij_ref[0, 0] = jnp.exp(coeff * (r - offs) ** 2).astype(jnp.bfloat16)
    c_ref[0, 0] = (r <= _CUTOFF).astype(jnp.bfloat16)


def _mega_kernel(x0_ref, fij_ref, c_ref, nbr_ref,
                 iw_ref, ib_ref, win_ref, bin_ref, wh_ref, bh_ref,
                 f2w_ref, f2b_ref, dw_ref, db_ref,
                 xo_ref, xall_ref, y_ref):
    i = pl.program_id(0)
    b = pl.program_id(1)
    gelu = jax.nn.gelu
    bf16 = jnp.bfloat16

    @pl.when(jnp.logical_and(i == 0, pl.program_id(2) == 0))
    def _init_x():
        xall_ref[b] = x0_ref[0]

    @pl.when(pl.program_id(2) == 0)
    def _compute_y():
        for cc in range(_NC):
            yc = jnp.dot(xall_ref[b, cc], iw_ref[0],
                         preferred_element_type=jnp.float32)
            y_ref[cc] = (yc + ib_ref[0]).astype(bf16)

    fij = fij_ref[0, 0]
    w = gelu((jnp.dot(fij, win_ref[0], preferred_element_type=jnp.float32)
              + bin_ref[0]).astype(bf16))
    for j in range(_NFB):
        w = gelu((jnp.dot(w, wh_ref[0, j], preferred_element_type=jnp.float32)
                  + bh_ref[0, j]).astype(bf16))
    w = w * c_ref[0, 0]
    nbr = nbr_ref[0, 0]
    iot = lax.broadcasted_iota(jnp.int32, (_E, _AP), 1)
    oh = (iot == nbr).astype(bf16)
    yj = jnp.dot(oh[:, 0:_CA], y_ref[0], preferred_element_type=jnp.float32)
    for cc in range(1, _NC):
        yj = yj + jnp.dot(oh[:, cc * _CA:(cc + 1) * _CA], y_ref[cc],
                          preferred_element_type=jnp.float32)
    prod = yj * w
    parts = [prod[k * _CA:(k + 1) * _CA, :] for k in range(_NBH)]
    while len(parts) > 1:
        parts = [parts[j] + parts[j + 1] for j in range(0, len(parts), 2)]
    yagg = parts[0]
    t = gelu(jnp.dot(yagg, f2w_ref[0], preferred_element_type=jnp.float32)
             + f2b_ref[0])
    v = jnp.dot(t, dw_ref[0], preferred_element_type=jnp.float32) + db_ref[0]
    xn = xall_ref[b, pl.program_id(2)] + v
    xall_ref[b, pl.program_id(2)] = xn
    xo_ref[0, 0] = xn


def _full(shape):
    return pl.BlockSpec(shape, lambda *_: tuple(0 for _ in shape))


def kernel(atomic_numbers, positions, cell, cell_offset, neighbors,
           neighbor_mask, atom_mask, emb, filt_Win, filt_bin, filt_Wh,
           filt_bh, in2f_W, in2f_b, f2out_W, f2out_b, dense_W, dense_b):
    f32, bf16 = jnp.float32, jnp.bfloat16
    pada = _AP - _A
    az = jnp.pad(atomic_numbers, ((0, 0), (0, pada))).astype(jnp.int32)[..., None]
    pos = jnp.pad(positions, ((0, 0), (0, pada), (0, 0)))
    phi = pos.astype(bf16)
    plo = (pos - phi.astype(f32)).astype(bf16)
    ptab = jnp.concatenate(
        [phi, jnp.zeros((_B, _AP, 1), bf16), plo,
         jnp.zeros((_B, _AP, _D - 7), bf16)], axis=-1)
    nbr = jnp.pad(neighbors, ((0, 0), (0, pada), (0, 0)))
    nbr_k = (nbr.reshape(_B, _NC, _CA, _NBH).transpose(0, 1, 3, 2)
             .reshape(_B, _NC, _E, 1).astype(jnp.int32))
    ehi16 = emb.astype(bf16)
    ehi = jnp.zeros((128, _D), bf16).at[:_MAXZ].set(ehi16)
    elo = jnp.zeros((128, _D), bf16).at[:_MAXZ].set(
        (emb - ehi16.astype(f32)).astype(bf16))
    winp = jnp.pad(filt_Win, ((0, 0), (0, _NGP - _NG), (0, 0))).astype(bf16)
    wh16 = filt_Wh.astype(bf16)

    x0 = pl.pallas_call(
        _emb_kernel, grid=(_B,),
        in_specs=[pl.BlockSpec((1, _AP, 1), lambda b: (b, 0, 0)),
                  _full((128, _D)), _full((128, _D))],
        out_specs=pl.BlockSpec((1, _AP, _D), lambda b: (b, 0, 0)),
        out_shape=jax.ShapeDtypeStruct((_B, _AP, _D), f32),
    )(az, ehi, elo)

    fij, cmask = pl.pallas_call(
        _fij_kernel, grid=(_B, _NC),
        in_specs=[pl.BlockSpec((1, 1, _E, 1), lambda b, c: (b, c, 0, 0)),
                  pl.BlockSpec((1, _AP, _D), lambda b, c: (b, 0, 0)),
                  pl.BlockSpec((1, _CA, _D), lambda b, c: (b, c, 0))],
        out_specs=[pl.BlockSpec((1, 1, _E, _NGP), lambda b, c: (b, c, 0, 0)),
                   pl.BlockSpec((1, 1, _E, 1), lambda b, c: (b, c, 0, 0))],
        out_shape=[jax.ShapeDtypeStruct((_B, _NC, _E, _NGP), bf16),
                   jax.ShapeDtypeStruct((_B, _NC, _E, 1), bf16)],
    )(nbr_k, ptab, ptab)

    x0_4d = x0.reshape(_B, _NC, _CA, _D)
    xf = pl.pallas_call(
        _mega_kernel, grid=(_NI, _B, _NC),
        in_specs=[
            pl.BlockSpec((1, _NC, _CA, _D), lambda i, b, c: (b, 0, 0, 0)),
            pl.BlockSpec((1, 1, _E, _NGP), lambda i, b, c: (b, c, 0, 0)),
            pl.BlockSpec((1, 1, _E, 1), lambda i, b, c: (b, c, 0, 0)),
            pl.BlockSpec((1, 1, _E, 1), lambda i, b, c: (b, c, 0, 0)),
            pl.BlockSpec((1, _D, _D), lambda i, b, c: (i, 0, 0)),
            pl.BlockSpec((1, 1, _D), lambda i, b, c: (i, 0, 0)),
            pl.BlockSpec((1, _NGP, _D), lambda i, b, c: (i, 0, 0)),
            pl.BlockSpec((1, 1, _D), lambda i, b, c: (i, 0, 0)),
            pl.BlockSpec((1, _NFB, _D, _D), lambda i, b, c: (i, 0, 0, 0)),
            pl.BlockSpec((1, _NFB, 1, _D), lambda i, b, c: (i, 0, 0, 0)),
            pl.BlockSpec((1, _D, _D), lambda i, b, c: (i, 0, 0)),
            pl.BlockSpec((1, 1, _D), lambda i, b, c: (i, 0, 0)),
            pl.BlockSpec((1, _D, _D), lambda i, b, c: (i, 0, 0)),
            pl.BlockSpec((1, 1, _D), lambda i, b, c: (i, 0, 0)),
        ],
        out_specs=pl.BlockSpec((1, 1, _CA, _D), lambda i, b, c: (b, c, 0, 0)),
        out_shape=jax.ShapeDtypeStruct((_B, _NC, _CA, _D), f32),
        scratch_shapes=[pltpu.VMEM((_B, _NC, _CA, _D), f32),
                        pltpu.VMEM((_NC, _CA, _D), bf16)],
        compiler_params=pltpu.CompilerParams(
            dimension_semantics=("arbitrary", "arbitrary", "arbitrary")),
    )(x0_4d, fij, cmask, nbr_k,
      in2f_W, in2f_b.reshape(_NI, 1, _D),
      winp, filt_bin.reshape(_NI, 1, _D),
      wh16, filt_bh.reshape(_NI, _NFB, 1, _D),
      f2out_W, f2out_b.reshape(_NI, 1, _D),
      dense_W, dense_b.reshape(_NI, 1, _D))

    return xf.reshape(_B, _AP, _D)[:, :_A, :]


# speedup vs baseline: 10.1960x; 1.0043x over previous
import jax
import jax.numpy as jnp
from jax import lax
from jax.experimental import pallas as pl
from jax.experimental.pallas import tpu as pltpu

_B, _A, _NBH = 10, 1000, 32
_D = 128
_NG = 25
_NI = 3
_NFB = 3
_CUTOFF = 5.0
_MAXZ = 100

_AP = 1024
_CA = 256
_NC = _AP // _CA
_E = _CA * _NBH
_NGP = 32


def _fij_kernel(nbr_ref, p_ref, pc_ref, fij_ref, c_ref):
    nbr = nbr_ref[0, 0]
    iot = lax.broadcasted_iota(jnp.int32, (_E, _AP), 1)
    oh = (iot == nbr).astype(jnp.bfloat16)
    dall = jnp.dot(oh, p_ref[0], preferred_element_type=jnp.float32)
    pc = pc_ref[0].astype(jnp.float32)
    dall = dall - jnp.concatenate([pc] * _NBH, axis=0)
    r2 = jnp.zeros((_E, 1), jnp.float32)
    for c in range(3):
        dv = dall[:, c:c + 1] + dall[:, c + 4:c + 5]
        r2 = r2 + dv * dv
    r = jnp.sqrt(r2)
    width = _CUTOFF / (_NG - 1)
    coeff = -0.5 / (width * width)
    offs = lax.broadcasted_iota(jnp.int32, (_E, _NGP), 1).astype(jnp.float32) * width
    fij_ref[0, 0] = jnp.exp(coeff * (r - offs) ** 2).astype(jnp.bfloat16)
    c_ref[0, 0] = (r <= _CUTOFF).astype(jnp.bfloat16)


def _mega_kernel(az_ref, ehi_ref, elo_ref, fij_ref, c_ref, nbr_ref,
                 iw_ref, ib_ref, win_ref, bin_ref, wh_ref, bh_ref,
                 f2w_ref, f2b_ref, dw_ref, db_ref,
                 xo_ref, xall_ref, y_ref):
    i = pl.program_id(0)
    b = pl.program_id(1)
    gelu = jax.nn.gelu
    bf16 = jnp.bfloat16

    @pl.when(jnp.logical_and(i == 0, pl.program_id(2) == 0))
    def _init_x():
        azi = az_ref[0]
        ziot = lax.broadcasted_iota(jnp.int32, (_AP, 128), 1)
        ohz = (ziot == azi).astype(bf16)
        xe = jnp.dot(ohz, ehi_ref[...], preferred_element_type=jnp.float32)
        xe = xe + jnp.dot(ohz, elo_ref[...], preferred_element_type=jnp.float32)
        for cc in range(_NC):
            xall_ref[b, cc] = xe[cc * _CA:(cc + 1) * _CA]

    @pl.when(pl.program_id(2) == 0)
    def _compute_y():
        for cc in range(_NC):
            yc = jnp.dot(xall_ref[b, cc], iw_ref[0],
                         preferred_element_type=jnp.float32)
            y_ref[cc] = (yc + ib_ref[0]).astype(bf16)

    fij = fij_ref[0, 0]
    w = gelu((jnp.dot(fij, win_ref[0], preferred_element_type=jnp.float32)
              + bin_ref[0]).astype(bf16))
    for j in range(_NFB):
        w = gelu((jnp.dot(w, wh_ref[0, j], preferred_element_type=jnp.float32)
                  + bh_ref[0, j]).astype(bf16))
    w = w * c_ref[0, 0]
    nbr = nbr_ref[0, 0]
    iot = lax.broadcasted_iota(jnp.int32, (_E, _AP), 1)
    oh = (iot == nbr).astype(bf16)
    yj = jnp.dot(oh[:, 0:_CA], y_ref[0], preferred_element_type=jnp.float32)
    for cc in range(1, _NC):
        yj = yj + jnp.dot(oh[:, cc * _CA:(cc + 1) * _CA], y_ref[cc],
                          preferred_element_type=jnp.float32)
    prod = yj * w
    parts = [prod[k * _CA:(k + 1) * _CA, :] for k in range(_NBH)]
    while len(parts) > 1:
        parts = [parts[j] + parts[j + 1] for j in range(0, len(parts), 2)]
    yagg = parts[0]
    t = gelu(jnp.dot(yagg, f2w_ref[0], preferred_element_type=jnp.float32)
             + f2b_ref[0])
    v = jnp.dot(t, dw_ref[0], preferred_element_type=jnp.float32) + db_ref[0]
    xn = xall_ref[b, pl.program_id(2)] + v
    xall_ref[b, pl.program_id(2)] = xn
    xo_ref[0, 0] = xn


def _full(shape):
    return pl.BlockSpec(shape, lambda *_: tuple(0 for _ in shape))


def kernel(atomic_numbers, positions, cell, cell_offset, neighbors,
           neighbor_mask, atom_mask, emb, filt_Win, filt_bin, filt_Wh,
           filt_bh, in2f_W, in2f_b, f2out_W, f2out_b, dense_W, dense_b):
    f32, bf16 = jnp.float32, jnp.bfloat16
    pada = _AP - _A
    az = jnp.pad(atomic_numbers, ((0, 0), (0, pada))).astype(jnp.int32)[..., None]
    pos = jnp.pad(positions, ((0, 0), (0, pada), (0, 0)))
    phi = pos.astype(bf16)
    plo = (pos - phi.astype(f32)).astype(bf16)
    ptab = jnp.concatenate(
        [phi, jnp.zeros((_B, _AP, 1), bf16), plo,
         jnp.zeros((_B, _AP, _D - 7), bf16)], axis=-1)
    nbr = jnp.pad(neighbors, ((0, 0), (0, pada), (0, 0)))
    nbr_k = (nbr.reshape(_B, _NC, _CA, _NBH).transpose(0, 1, 3, 2)
             .reshape(_B, _NC, _E, 1).astype(jnp.int32))
    ehi16 = emb.astype(bf16)
    ehi = jnp.zeros((128, _D), bf16).at[:_MAXZ].set(ehi16)
    elo = jnp.zeros((128, _D), bf16).at[:_MAXZ].set(
        (emb - ehi16.astype(f32)).astype(bf16))
    winp = jnp.pad(filt_Win, ((0, 0), (0, _NGP - _NG), (0, 0))).astype(bf16)
    wh16 = filt_Wh.astype(bf16)

    fij, cmask = pl.pallas_call(
        _fij_kernel, grid=(_B, _NC),
        in_specs=[pl.BlockSpec((1, 1, _E, 1), lambda b, c: (b, c, 0, 0)),
                  pl.BlockSpec((1, _AP, _D), lambda b, c: (b, 0, 0)),
                  pl.BlockSpec((1, _CA, _D), lambda b, c: (b, c, 0))],
        out_specs=[pl.BlockSpec((1, 1, _E, _NGP), lambda b, c: (b, c, 0, 0)),
                   pl.BlockSpec((1, 1, _E, 1), lambda b, c: (b, c, 0, 0))],
        out_shape=[jax.ShapeDtypeStruct((_B, _NC, _E, _NGP), bf16),
                   jax.ShapeDtypeStruct((_B, _NC, _E, 1), bf16)],
    )(nbr_k, ptab, ptab)

    xf = pl.pallas_call(
        _mega_kernel, grid=(_NI, _B, _NC),
        in_specs=[
            pl.BlockSpec((1, _AP, 1), lambda i, b, c: (b, 0, 0)),
            pl.BlockSpec((128, _D), lambda i, b, c: (0, 0)),
            pl.BlockSpec((128, _D), lambda i, b, c: (0, 0)),
            pl.BlockSpec((1, 1, _E, _NGP), lambda i, b, c: (b, c, 0, 0)),
            pl.BlockSpec((1, 1, _E, 1), lambda i, b, c: (b, c, 0, 0)),
            pl.BlockSpec((1, 1, _E, 1), lambda i, b, c: (b, c, 0, 0)),
            pl.BlockSpec((1, _D, _D), lambda i, b, c: (i, 0, 0)),
            pl.BlockSpec((1, 1, _D), lambda i, b, c: (i, 0, 0)),
            pl.BlockSpec((1, _NGP, _D), lambda i, b, c: (i, 0, 0)),
            pl.BlockSpec((1, 1, _D), lambda i, b, c: (i, 0, 0)),
            pl.BlockSpec((1, _NFB, _D, _D), lambda i, b, c: (i, 0, 0, 0)),
            pl.BlockSpec((1, _NFB, 1, _D), lambda i, b, c: (i, 0, 0, 0)),
            pl.BlockSpec((1, _D, _D), lambda i, b, c: (i, 0, 0)),
            pl.BlockSpec((1, 1, _D), lambda i, b, c: (i, 0, 0)),
            pl.BlockSpec((1, _D, _D), lambda i, b, c: (i, 0, 0)),
            pl.BlockSpec((1, 1, _D), lambda i, b, c: (i, 0, 0)),
        ],
        out_specs=pl.BlockSpec((1, 1, _CA, _D), lambda i, b, c: (b, c, 0, 0)),
        out_shape=jax.ShapeDtypeStruct((_B, _NC, _CA, _D), f32),
        scratch_shapes=[pltpu.VMEM((_B, _NC, _CA, _D), f32),
                        pltpu.VMEM((_NC, _CA, _D), bf16)],
        compiler_params=pltpu.CompilerParams(
            dimension_semantics=("arbitrary", "arbitrary", "arbitrary")),
    )(az, ehi, elo, fij, cmask, nbr_k,
      in2f_W, in2f_b.reshape(_NI, 1, _D),
      winp, filt_bin.reshape(_NI, 1, _D),
      wh16, filt_bh.reshape(_NI, _NFB, 1, _D),
      f2out_W, f2out_b.reshape(_NI, 1, _D),
      dense_W, dense_b.reshape(_NI, 1, _D))

    return xf.reshape(_B, _AP, _D)[:, :_A, :]
